# Initial kernel scaffold; baseline (speedup 1.0000x reference)
import jax
import jax.numpy as jnp
from jax.experimental import pallas as pl


def kernel(x, table, W, b):
    raise NotImplementedError("write your pallas kernel here")


# speedup vs baseline: 1.7295x; 1.7295x over previous
import functools

import jax
import jax.numpy as jnp
from jax import lax
from jax.experimental import pallas as pl
from jax.experimental.pallas import tpu as pltpu
from jax.experimental.pallas import tpu_sc as plsc

VOCAB = 1000000
EMB = 64
OUT = 128
B = 16384
L = 200

NC = 2
NS = 16
NW = NC * NS
BLK = 128
KPW = B // (NW * BLK)
LPAD = L + 2

_mesh = plsc.VectorSubcoreMesh(
    core_axis_name="c", subcore_axis_name="s", num_cores=NC, num_subcores=NS
)


@functools.partial(
    pl.kernel,
    out_type=jax.ShapeDtypeStruct((B, EMB), jnp.float32),
    mesh=_mesh,
    scratch_types=[
        pltpu.VMEM((LPAD, BLK), jnp.int32),
        pltpu.VMEM((2, BLK, EMB), jnp.float32),
        pltpu.VMEM((BLK, EMB), jnp.float32),
        pltpu.SemaphoreType.DMA,
        pltpu.SemaphoreType.DMA,
    ],
    compiler_params=pltpu.CompilerParams(use_tc_tiling_on_sc=False),
)
def _pooled_sums(xt_hbm, table_hbm, out_hbm, idx_v, rows_v, accum_v, sem0, sem1):
    wid = lax.axis_index("s") * NC + lax.axis_index("c")
    sems = (sem0, sem1)

    for k in range(KPW):
        blk = wid * KPW + k
        pltpu.sync_copy(xt_hbm.at[blk], idx_v)

        pltpu.async_copy(table_hbm.at[idx_v.at[0]], rows_v.at[0], sem0)
        pltpu.async_copy(table_hbm.at[idx_v.at[1]], rows_v.at[1], sem1)

        def _zero(i, carry):
            zero = jnp.zeros((16,), jnp.float32)
            for cc in range(EMB // 16):
                accum_v[i, pl.ds(cc * 16, 16)] = zero
            return carry

        lax.fori_loop(0, BLK, _zero, 0, unroll=8)

        def _pair(m, carry):
            for bb in range(2):
                pltpu.make_async_copy(
                    table_hbm.at[pl.ds(0, BLK)], rows_v.at[bb], sems[bb]
                ).wait()
                def _acc(i, c2):
                    for cc in range(EMB // 16):
                        plsc.addupdate(
                            accum_v.at[i, pl.ds(cc * 16, 16)],
                            rows_v[bb, i, pl.ds(cc * 16, 16)],
                        )
                    return c2

                lax.fori_loop(0, BLK, _acc, 0, unroll=8)
                pltpu.async_copy(
                    table_hbm.at[idx_v.at[2 * m + 2 + bb]], rows_v.at[bb], sems[bb]
                )
            return carry

        lax.fori_loop(0, L // 2, _pair, 0)

        pltpu.make_async_copy(table_hbm.at[pl.ds(0, BLK)], rows_v.at[0], sem0).wait()
        pltpu.make_async_copy(table_hbm.at[pl.ds(0, BLK)], rows_v.at[1], sem1).wait()

        pltpu.sync_copy(accum_v, out_hbm.at[pl.ds(blk * BLK, BLK)])


def _project(pooled, W, b):
    BS = 1024

    def body(p_ref, w_ref, b_ref, o_ref):
        o_ref[...] = (
            jnp.dot(p_ref[...], w_ref[...], preferred_element_type=jnp.float32)
            * (1.0 / L)
            + b_ref[...]
        )

    return pl.pallas_call(
        body,
        grid=(B // BS,),
        in_specs=[
            pl.BlockSpec((BS, EMB), lambda i: (i, 0)),
            pl.BlockSpec((EMB, OUT), lambda i: (0, 0)),
            pl.BlockSpec((1, OUT), lambda i: (0, 0)),
        ],
        out_specs=pl.BlockSpec((BS, OUT), lambda i: (i, 0)),
        out_shape=jax.ShapeDtypeStruct((B, OUT), jnp.float32),
    )(pooled, W, b.reshape(1, OUT))


def kernel(x, table, W, b):
    x32 = x.astype(jnp.int32)
    xt = x32.reshape(NW * KPW, BLK, L).transpose(0, 2, 1)
    xt = jnp.pad(xt, ((0, 0), (0, 2), (0, 0)))
    pooled = _pooled_sums(xt, table)
    return _project(pooled, W, b)
